# baseline (speedup 1.0000x reference)
import functools

import jax
import jax.numpy as jnp
import numpy as np
from jax import lax
from jax.experimental import pallas as pl
from jax.experimental.pallas import tpu as pltpu
from jax.experimental.pallas import tpu_sc as plsc

N_HEADS = 16
HEAD_DIM = 128
ROPE_DIM = 64
TOPK = 256
EPS = 1e-6

ROWS = 2048
PREP_TILE = 256
SCORE_TILE = 64

CAP = 272
SLACK = CAP + 16
NW = 32
RPW = ROWS // NW


def _hada128():
    H = np.array([[1.0]], dtype=np.float32)
    while H.shape[0] < HEAD_DIM:
        H = np.block([[H, H], [H, -H]]).astype(np.float32)
    return H * (HEAD_DIM ** -0.5)


def _bf(v):
    return v.astype(jnp.bfloat16)


def _qprep_body(qr_ref, rs_ref, wqbT_ref, hb_ref, qh_ref):
    q = jnp.dot(_bf(qr_ref[...]), wqbT_ref[...], preferred_element_type=jnp.float32)
    rs = rs_ref[...]
    scale_head = jnp.concatenate([rs, jnp.ones_like(rs)], axis=-1)
    for h in range(N_HEADS):
        qs = q[:, h * HEAD_DIM:(h + 1) * HEAD_DIM] * scale_head
        qh_ref[:, h * HEAD_DIM:(h + 1) * HEAD_DIM] = jnp.dot(
            _bf(qs), hb_ref[...], preferred_element_type=jnp.float32)


def _score_body(qh_ref, khT_ref, w_ref, out_ref):
    w = w_ref[...]
    acc = None
    for h in range(N_HEADS):
        s = jnp.dot(qh_ref[h], khT_ref[...], preferred_element_type=jnp.float32)
        s = jnp.maximum(s * (HEAD_DIM ** -0.5), 0.0) * w[:, h:h + 1]
        acc = s if acc is None else acc + s
    out_ref[...] = acc


def _scan_level(hist_ref, ghist_ref, target):
    def gbody(i, car):
        s, g1, sg = car
        g = 15 - i
        c = jnp.sum(ghist_ref[pl.ds(g * 16, 16)])
        s2 = s + c
        hit = jnp.logical_and(s < target, s2 >= target)
        return (s2, jnp.where(hit, g, g1), jnp.where(hit, s, sg))

    _, g1, sg = lax.fori_loop(0, 16, gbody, (jnp.int32(0), jnp.int32(0), jnp.int32(0)))

    def bbody(i, car):
        s, b1, sb = car
        b = g1 * 16 + 15 - i
        c = jnp.sum(hist_ref[pl.ds(b * 16, 16)])
        s2 = s + c
        hit = jnp.logical_and(s < target, s2 >= target)
        return (s2, jnp.where(hit, b, b1), jnp.where(hit, s, sb))

    _, b1, sb = lax.fori_loop(0, 16, bbody, (sg, jnp.int32(0), jnp.int32(0)))
    return b1, sb


def _sel_body(scores_hbm, outk_hbm, outi_hbm,
              bufA, bufB, kb, h1, hg1, h2, hg2,
              skA, siA, skB, siB, semA, semB, semOA, semOB):
    cid = lax.axis_index("c")
    sid = lax.axis_index("s")
    wid = sid * 2 + cid
    r0 = wid * RPW
    lane = lax.broadcasted_iota(jnp.int32, (16,), 0)
    onesv = jnp.ones((16,), jnp.int32)
    zerov = jnp.zeros((16,), jnp.int32)
    KEYMIN = jnp.int32(-2147483647)

    def process_row(buf, sk, si, semO, r):
        def zh(i, _):
            for u in range(8):
                h1[pl.ds((i * 8 + u) * 16, 16)] = zerov
                h2[pl.ds((i * 8 + u) * 16, 16)] = zerov
            return 0
        lax.fori_loop(0, 32, zh, 0)
        for u in range(16):
            hg1[pl.ds(u * 16, 16)] = zerov
            hg2[pl.ds(u * 16, 16)] = zerov

        @pl.when(r - r0 >= 2)
        def _():
            pltpu.make_async_copy(sk.at[pl.ds(0, CAP)], outk_hbm.at[pl.ds(r * CAP, CAP)], semO).wait()
            pltpu.make_async_copy(si.at[pl.ds(0, CAP)], outi_hbm.at[pl.ds(r * CAP, CAP)], semO).wait()

        def zp(i, _):
            for u in range(3):
                sk[pl.ds((i * 3 + u) * 16, 16)] = jnp.full((16,), KEYMIN, jnp.int32)
                si[pl.ds((i * 3 + u) * 16, 16)] = zerov
            return 0
        lax.fori_loop(0, SLACK // 48, zp, 0)

        def p1(i, _):
            for u in range(4):
                o = (i * 4 + u) * 16
                v = buf[pl.ds(o, 16)]
                bi = lax.bitcast_convert_type(v, jnp.int32)
                key = bi ^ (lax.shift_right_arithmetic(bi, 31) & jnp.int32(0x7FFFFFFF))
                kb[pl.ds(o, 16)] = key
                cb = lax.shift_right_arithmetic(key, 24) + 128
                plsc.addupdate_scatter(h1, [cb * 16 + lane], onesv)
                plsc.addupdate_scatter(hg1, [lax.shift_right_logical(cb, 4) * 16 + lane], onesv)
            return 0
        lax.fori_loop(0, 32, p1, 0)

        b1, nab1 = _scan_level(h1, hg1, jnp.int32(TOPK))

        def p2(i, _):
            for u in range(4):
                o = (i * 4 + u) * 16
                key = kb[pl.ds(o, 16)]
                cb = lax.shift_right_arithmetic(key, 24) + 128
                m = cb == b1
                fb = lax.shift_right_arithmetic(key, 16) & jnp.int32(0xFF)
                plsc.addupdate_scatter(h2, [fb * 16 + lane], onesv, mask=m)
                plsc.addupdate_scatter(hg2, [lax.shift_right_logical(fb, 4) * 16 + lane], onesv, mask=m)
            return 0
        lax.fori_loop(0, 32, p2, 0)

        b2, _nab2 = _scan_level(h2, hg2, jnp.int32(TOPK) - nab1)
        thr = lax.shift_left(b1 - 128, 24) + lax.shift_left(b2, 16)

        def p3(i, wp):
            for u in range(4):
                o = (i * 4 + u) * 16
                key = kb[pl.ds(o, 16)]
                m = key >= thr
                pref = plsc.cumsum(jnp.where(m, 1, 0).astype(jnp.int32))
                pos = jnp.maximum(wp + pref - 1, 0)
                m2 = jnp.logical_and(m, pos < CAP)
                plsc.store_scatter(sk, [pos], key, mask=m2)
                plsc.store_scatter(si, [pos], o + lane, mask=m2)
                wp = jnp.minimum(wp + jnp.max(pref), jnp.int32(CAP))
            return wp
        lax.fori_loop(0, 32, p3, jnp.int32(0))

        pltpu.make_async_copy(sk.at[pl.ds(0, CAP)], outk_hbm.at[pl.ds(r * CAP, CAP)], semO).start()
        pltpu.make_async_copy(si.at[pl.ds(0, CAP)], outi_hbm.at[pl.ds(r * CAP, CAP)], semO).start()

    pltpu.make_async_copy(scores_hbm.at[pl.ds(r0 * ROWS, ROWS)], bufA, semA).start()

    def pair(rp, _):
        rA = r0 + 2 * rp
        rB = rA + 1
        pltpu.make_async_copy(scores_hbm.at[pl.ds(rB * ROWS, ROWS)], bufB, semB).start()
        pltpu.make_async_copy(scores_hbm.at[pl.ds(rA * ROWS, ROWS)], bufA, semA).wait()
        process_row(bufA, skA, siA, semOA, rA)

        @pl.when(rp < RPW // 2 - 1)
        def _():
            pltpu.make_async_copy(scores_hbm.at[pl.ds((rA + 2) * ROWS, ROWS)], bufA, semA).start()

        pltpu.make_async_copy(scores_hbm.at[pl.ds(rB * ROWS, ROWS)], bufB, semB).wait()
        process_row(bufB, skB, siB, semOB, rB)
        return 0

    lax.fori_loop(0, RPW // 2, pair, 0)

    rlast = r0 + RPW - 2
    pltpu.make_async_copy(skA.at[pl.ds(0, CAP)], outk_hbm.at[pl.ds(rlast * CAP, CAP)], semOA).wait()
    pltpu.make_async_copy(siA.at[pl.ds(0, CAP)], outi_hbm.at[pl.ds(rlast * CAP, CAP)], semOA).wait()
    pltpu.make_async_copy(skB.at[pl.ds(0, CAP)], outk_hbm.at[pl.ds((rlast + 1) * CAP, CAP)], semOB).wait()
    pltpu.make_async_copy(siB.at[pl.ds(0, CAP)], outi_hbm.at[pl.ds((rlast + 1) * CAP, CAP)], semOB).wait()


def _select_candidates(index_scores):
    mesh = plsc.VectorSubcoreMesh(core_axis_name="c", subcore_axis_name="s")
    f = pl.kernel(
        _sel_body,
        out_type=[
            jax.ShapeDtypeStruct((ROWS * CAP,), jnp.int32),
            jax.ShapeDtypeStruct((ROWS * CAP,), jnp.int32),
        ],
        mesh=mesh,
        scratch_types=[
            pltpu.VMEM((ROWS,), jnp.float32),
            pltpu.VMEM((ROWS,), jnp.float32),
            pltpu.VMEM((ROWS,), jnp.int32),
            pltpu.VMEM((4096,), jnp.int32),
            pltpu.VMEM((256,), jnp.int32),
            pltpu.VMEM((4096,), jnp.int32),
            pltpu.VMEM((256,), jnp.int32),
            pltpu.VMEM((SLACK,), jnp.int32),
            pltpu.VMEM((SLACK,), jnp.int32),
            pltpu.VMEM((SLACK,), jnp.int32),
            pltpu.VMEM((SLACK,), jnp.int32),
            pltpu.SemaphoreType.DMA,
            pltpu.SemaphoreType.DMA,
            pltpu.SemaphoreType.DMA,
            pltpu.SemaphoreType.DMA,
        ],
        compiler_params=pltpu.CompilerParams(needs_layout_passes=False),
    )
    return f(index_scores)


def _rank_body(kb_ref, tri_ref, rank_ref):
    k3 = kb_ref[...]
    a = k3[:, :, None]
    b = k3[:, None, :]
    gt = b > a
    tie = jnp.logical_and(a == b, tri_ref[...][None, :, :] > 0)
    rank_ref[...] = jnp.sum(jnp.logical_or(gt, tie).astype(jnp.int32), axis=2)


def _place_body(ranks_hbm, ci_hbm, out_hbm, rbuf, ibuf, stage, sem):
    cid = lax.axis_index("c")
    sid = lax.axis_index("s")
    wid = sid * 2 + cid
    r0 = wid * RPW
    BR = 8

    def batch(bb, _):
        r = r0 + bb * BR
        pltpu.sync_copy(ranks_hbm.at[pl.ds(r * CAP, BR * CAP)], rbuf)
        pltpu.sync_copy(ci_hbm.at[pl.ds(r * CAP, BR * CAP)], ibuf)

        def v(i, _):
            rank = rbuf[pl.ds(i * 16, 16)]
            idx = ibuf[pl.ds(i * 16, 16)]
            rowu = lax.div(i, jnp.int32(CAP // 16))
            m = rank < TOPK
            tgt = jnp.where(m, rank, 0) + rowu * TOPK
            plsc.store_scatter(stage, [tgt], idx, mask=m)
            return 0
        lax.fori_loop(0, BR * CAP // 16, v, 0)
        pltpu.sync_copy(stage, out_hbm.at[pl.ds(r * TOPK, BR * TOPK)])
        return 0

    lax.fori_loop(0, RPW // BR, batch, 0)


def _place(ranks, ci):
    mesh = plsc.VectorSubcoreMesh(core_axis_name="c", subcore_axis_name="s")
    f = pl.kernel(
        _place_body,
        out_type=jax.ShapeDtypeStruct((ROWS * TOPK,), jnp.int32),
        mesh=mesh,
        scratch_types=[
            pltpu.VMEM((8 * CAP,), jnp.int32),
            pltpu.VMEM((8 * CAP,), jnp.int32),
            pltpu.VMEM((8 * TOPK,), jnp.int32),
            pltpu.SemaphoreType.DMA,
        ],
        compiler_params=pltpu.CompilerParams(needs_layout_passes=False),
    )
    return f(ranks, ci)


def kernel(x, qr, freqs_cis, wq_b, wk, k_norm_w, k_norm_b, weights_proj):
    b, s, _ = x.shape
    qr2 = qr[0]
    rs = jnp.concatenate([freqs_cis, freqs_cis], axis=-1)
    Hb = _bf(jnp.asarray(_hada128()))
    wqbT = _bf(wq_b.T)

    kx = (x @ wk.T).astype(jnp.float32)
    mu = jnp.mean(kx, axis=-1, keepdims=True)
    var = jnp.mean((kx - mu) ** 2, axis=-1, keepdims=True)
    k = (kx - mu) / jnp.sqrt(var + EPS) * k_norm_w + k_norm_b
    k = k * jnp.concatenate([rs, jnp.ones_like(rs)], axis=-1)[None]
    kh = k[0] @ jnp.asarray(_hada128())
    weights = (x @ weights_proj.T)[0] * (N_HEADS ** -0.5)
    khT = _bf(kh.T)

    n_prep = ROWS // PREP_TILE
    qh = pl.pallas_call(
        _qprep_body,
        grid=(n_prep,),
        in_specs=[
            pl.BlockSpec((PREP_TILE, 512), lambda i: (i, 0)),
            pl.BlockSpec((PREP_TILE, ROPE_DIM), lambda i: (i, 0)),
            pl.BlockSpec((512, 2048), lambda i: (0, 0)),
            pl.BlockSpec((HEAD_DIM, HEAD_DIM), lambda i: (0, 0)),
        ],
        out_specs=pl.BlockSpec((PREP_TILE, 2048), lambda i: (i, 0)),
        out_shape=jax.ShapeDtypeStruct((ROWS, 2048), jnp.float32),
    )(qr2, rs, wqbT, Hb)

    qh_hm = _bf(qh.reshape(ROWS, N_HEADS, HEAD_DIM).transpose(1, 0, 2))

    n_sc = ROWS // SCORE_TILE
    index_scores = pl.pallas_call(
        _score_body,
        grid=(n_sc,),
        in_specs=[
            pl.BlockSpec((N_HEADS, SCORE_TILE, HEAD_DIM), lambda i: (0, i, 0)),
            pl.BlockSpec((HEAD_DIM, ROWS), lambda i: (0, 0)),
            pl.BlockSpec((SCORE_TILE, N_HEADS), lambda i: (i, 0)),
        ],
        out_specs=pl.BlockSpec((SCORE_TILE, ROWS), lambda i: (i, 0)),
        out_shape=jax.ShapeDtypeStruct((ROWS, ROWS), jnp.float32),
    )(qh_hm, khT, weights)

    cand_key_f, cand_idx_f = _select_candidates(index_scores.reshape(-1))
    cand_key = cand_key_f.reshape(ROWS, CAP)

    tri = jnp.tril(jnp.ones((CAP, CAP), jnp.int32), k=-1)
    ranks = pl.pallas_call(
        _rank_body,
        grid=(ROWS // 8,),
        in_specs=[
            pl.BlockSpec((8, CAP), lambda i: (i, 0)),
            pl.BlockSpec((CAP, CAP), lambda i: (0, 0)),
        ],
        out_specs=pl.BlockSpec((8, CAP), lambda i: (i, 0)),
        out_shape=jax.ShapeDtypeStruct((ROWS, CAP), jnp.int32),
    )(cand_key, tri)

    out = _place(ranks.reshape(-1), cand_idx_f)
    return out.reshape(1, ROWS, TOPK)


# speedup vs baseline: 1.0483x; 1.0483x over previous
import functools

import jax
import jax.numpy as jnp
import numpy as np
from jax import lax
from jax.experimental import pallas as pl
from jax.experimental.pallas import tpu as pltpu
from jax.experimental.pallas import tpu_sc as plsc

N_HEADS = 16
HEAD_DIM = 128
ROPE_DIM = 64
TOPK = 256
EPS = 1e-6

ROWS = 2048
PREP_TILE = 256
SCORE_TILE = 64

CAP = 272
SLACK = CAP + 16
NW = 32
RPW = ROWS // NW


def _hada128():
    H = np.array([[1.0]], dtype=np.float32)
    while H.shape[0] < HEAD_DIM:
        H = np.block([[H, H], [H, -H]]).astype(np.float32)
    return H * (HEAD_DIM ** -0.5)


def _bf(v):
    return v.astype(jnp.bfloat16)


def _qprep_body(qr_ref, rs_ref, wqbT_ref, hb_ref, qh_ref):
    q = jnp.dot(_bf(qr_ref[...]), wqbT_ref[...], preferred_element_type=jnp.float32)
    rs = rs_ref[...]
    scale_head = jnp.concatenate([rs, jnp.ones_like(rs)], axis=-1)
    for h in range(N_HEADS):
        qs = q[:, h * HEAD_DIM:(h + 1) * HEAD_DIM] * scale_head
        qh_ref[:, h * HEAD_DIM:(h + 1) * HEAD_DIM] = jnp.dot(
            _bf(qs), hb_ref[...], preferred_element_type=jnp.float32)


def _score_body(qh_ref, khT_ref, w_ref, out_ref):
    w = w_ref[...]
    acc = None
    for h in range(N_HEADS):
        s = jnp.dot(qh_ref[h], khT_ref[...], preferred_element_type=jnp.float32)
        s = jnp.maximum(s * (HEAD_DIM ** -0.5), 0.0) * w[:, h:h + 1]
        acc = s if acc is None else acc + s
    out_ref[...] = acc


def _scan_level(hist_ref, ghist_ref, target):
    def gbody(i, car):
        s, g1, sg = car
        g = 15 - i
        c = jnp.sum(ghist_ref[pl.ds(g * 16, 16)])
        s2 = s + c
        hit = jnp.logical_and(s < target, s2 >= target)
        return (s2, jnp.where(hit, g, g1), jnp.where(hit, s, sg))

    _, g1, sg = lax.fori_loop(0, 16, gbody, (jnp.int32(0), jnp.int32(0), jnp.int32(0)))

    def bbody(i, car):
        s, b1, sb = car
        b = g1 * 16 + 15 - i
        c = jnp.sum(hist_ref[pl.ds(b * 16, 16)])
        s2 = s + c
        hit = jnp.logical_and(s < target, s2 >= target)
        return (s2, jnp.where(hit, b, b1), jnp.where(hit, s, sb))

    _, b1, sb = lax.fori_loop(0, 16, bbody, (sg, jnp.int32(0), jnp.int32(0)))
    return b1, sb


def _sel_body(scores_hbm, outk_hbm, outi_hbm,
              bufA, bufB, kb, h1, hg1, h2, hg2,
              skA, siA, skB, siB, semA, semB, semOA, semOB):
    cid = lax.axis_index("c")
    sid = lax.axis_index("s")
    wid = sid * 2 + cid
    r0 = wid * RPW
    lane = lax.broadcasted_iota(jnp.int32, (16,), 0)
    onesv = jnp.ones((16,), jnp.int32)
    zerov = jnp.zeros((16,), jnp.int32)
    KEYMIN = jnp.int32(-2147483647)

    def process_row(buf, sk, si, semO, r):
        def zh(i, _):
            for u in range(8):
                h1[pl.ds((i * 8 + u) * 16, 16)] = zerov
                h2[pl.ds((i * 8 + u) * 16, 16)] = zerov
            return 0
        lax.fori_loop(0, 32, zh, 0)
        for u in range(16):
            hg1[pl.ds(u * 16, 16)] = zerov
            hg2[pl.ds(u * 16, 16)] = zerov

        @pl.when(r - r0 >= 2)
        def _():
            pltpu.make_async_copy(sk.at[pl.ds(0, CAP)], outk_hbm.at[pl.ds(r * CAP, CAP)], semO).wait()
            pltpu.make_async_copy(si.at[pl.ds(0, CAP)], outi_hbm.at[pl.ds(r * CAP, CAP)], semO).wait()

        def zp(i, _):
            for u in range(3):
                sk[pl.ds((i * 3 + u) * 16, 16)] = jnp.full((16,), KEYMIN, jnp.int32)
                si[pl.ds((i * 3 + u) * 16, 16)] = zerov
            return 0
        lax.fori_loop(0, SLACK // 48, zp, 0)

        def p1(i, _):
            for u in range(2):
                o = (i * 2 + u) * 16
                v = buf[pl.ds(o, 16)]
                bi = lax.bitcast_convert_type(v, jnp.int32)
                key = bi ^ (lax.shift_right_arithmetic(bi, 31) & jnp.int32(0x7FFFFFFF))
                kb[pl.ds(o, 16)] = key
                cb = lax.shift_right_arithmetic(key, 24) + 128
                plsc.addupdate_scatter(h1, [cb * 16 + lane], onesv)
                plsc.addupdate_scatter(hg1, [lax.shift_right_logical(cb, 4) * 16 + lane], onesv)
            return 0
        lax.fori_loop(0, 64, p1, 0)

        b1, nab1 = _scan_level(h1, hg1, jnp.int32(TOPK))

        def p2(i, _):
            for u in range(2):
                o = (i * 2 + u) * 16
                key = kb[pl.ds(o, 16)]
                cb = lax.shift_right_arithmetic(key, 24) + 128
                m = cb == b1
                fb = lax.shift_right_arithmetic(key, 16) & jnp.int32(0xFF)
                plsc.addupdate_scatter(h2, [fb * 16 + lane], onesv, mask=m)
                plsc.addupdate_scatter(hg2, [lax.shift_right_logical(fb, 4) * 16 + lane], onesv, mask=m)
            return 0
        lax.fori_loop(0, 64, p2, 0)

        b2, _nab2 = _scan_level(h2, hg2, jnp.int32(TOPK) - nab1)
        thr = lax.shift_left(b1 - 128, 24) + lax.shift_left(b2, 16)

        def p3(i, wp):
            for u in range(2):
                o = (i * 2 + u) * 16
                key = kb[pl.ds(o, 16)]
                m = key >= thr
                pref = plsc.cumsum(jnp.where(m, 1, 0).astype(jnp.int32))
                pos = jnp.maximum(wp + pref - 1, 0)
                m2 = jnp.logical_and(m, pos < CAP)
                plsc.store_scatter(sk, [pos], key, mask=m2)
                plsc.store_scatter(si, [pos], o + lane, mask=m2)
                wp = jnp.minimum(wp + jnp.max(pref), jnp.int32(CAP))
            return wp
        lax.fori_loop(0, 64, p3, jnp.int32(0))

        pltpu.make_async_copy(sk.at[pl.ds(0, CAP)], outk_hbm.at[pl.ds(r * CAP, CAP)], semO).start()
        pltpu.make_async_copy(si.at[pl.ds(0, CAP)], outi_hbm.at[pl.ds(r * CAP, CAP)], semO).start()

    pltpu.make_async_copy(scores_hbm.at[pl.ds(r0 * ROWS, ROWS)], bufA, semA).start()

    def pair(rp, _):
        rA = r0 + 2 * rp
        rB = rA + 1
        pltpu.make_async_copy(scores_hbm.at[pl.ds(rB * ROWS, ROWS)], bufB, semB).start()
        pltpu.make_async_copy(scores_hbm.at[pl.ds(rA * ROWS, ROWS)], bufA, semA).wait()
        process_row(bufA, skA, siA, semOA, rA)

        @pl.when(rp < RPW // 2 - 1)
        def _():
            pltpu.make_async_copy(scores_hbm.at[pl.ds((rA + 2) * ROWS, ROWS)], bufA, semA).start()

        pltpu.make_async_copy(scores_hbm.at[pl.ds(rB * ROWS, ROWS)], bufB, semB).wait()
        process_row(bufB, skB, siB, semOB, rB)
        return 0

    lax.fori_loop(0, RPW // 2, pair, 0)

    rlast = r0 + RPW - 2
    pltpu.make_async_copy(skA.at[pl.ds(0, CAP)], outk_hbm.at[pl.ds(rlast * CAP, CAP)], semOA).wait()
    pltpu.make_async_copy(siA.at[pl.ds(0, CAP)], outi_hbm.at[pl.ds(rlast * CAP, CAP)], semOA).wait()
    pltpu.make_async_copy(skB.at[pl.ds(0, CAP)], outk_hbm.at[pl.ds((rlast + 1) * CAP, CAP)], semOB).wait()
    pltpu.make_async_copy(siB.at[pl.ds(0, CAP)], outi_hbm.at[pl.ds((rlast + 1) * CAP, CAP)], semOB).wait()


def _select_candidates(index_scores):
    mesh = plsc.VectorSubcoreMesh(core_axis_name="c", subcore_axis_name="s")
    f = pl.kernel(
        _sel_body,
        out_type=[
            jax.ShapeDtypeStruct((ROWS * CAP,), jnp.int32),
            jax.ShapeDtypeStruct((ROWS * CAP,), jnp.int32),
        ],
        mesh=mesh,
        scratch_types=[
            pltpu.VMEM((ROWS,), jnp.float32),
            pltpu.VMEM((ROWS,), jnp.float32),
            pltpu.VMEM((ROWS,), jnp.int32),
            pltpu.VMEM((4096,), jnp.int32),
            pltpu.VMEM((256,), jnp.int32),
            pltpu.VMEM((4096,), jnp.int32),
            pltpu.VMEM((256,), jnp.int32),
            pltpu.VMEM((SLACK,), jnp.int32),
            pltpu.VMEM((SLACK,), jnp.int32),
            pltpu.VMEM((SLACK,), jnp.int32),
            pltpu.VMEM((SLACK,), jnp.int32),
            pltpu.SemaphoreType.DMA,
            pltpu.SemaphoreType.DMA,
            pltpu.SemaphoreType.DMA,
            pltpu.SemaphoreType.DMA,
        ],
        compiler_params=pltpu.CompilerParams(needs_layout_passes=False),
    )
    return f(index_scores)


def _rank_body(kb_ref, tri_ref, rank_ref):
    k3 = kb_ref[...]
    a = k3[:, :, None]
    b = k3[:, None, :]
    gt = a > b
    tie = jnp.logical_and(a == b, tri_ref[...][None, :, :] > 0)
    rank_ref[...] = jnp.sum(jnp.logical_or(gt, tie).astype(jnp.int32), axis=1)


def _place_body(ranks_hbm, ci_hbm, out_hbm, rbuf, ibuf, stage, sem):
    cid = lax.axis_index("c")
    sid = lax.axis_index("s")
    wid = sid * 2 + cid
    r0 = wid * RPW
    BR = 8

    def batch(bb, _):
        r = r0 + bb * BR
        pltpu.sync_copy(ranks_hbm.at[pl.ds(r * CAP, BR * CAP)], rbuf)
        pltpu.sync_copy(ci_hbm.at[pl.ds(r * CAP, BR * CAP)], ibuf)

        def v(i, _):
            rank = rbuf[pl.ds(i * 16, 16)]
            idx = ibuf[pl.ds(i * 16, 16)]
            rowu = lax.div(i, jnp.int32(CAP // 16))
            m = rank < TOPK
            tgt = jnp.where(m, rank, 0) + rowu * TOPK
            plsc.store_scatter(stage, [tgt], idx, mask=m)
            return 0
        lax.fori_loop(0, BR * CAP // 16, v, 0)
        pltpu.sync_copy(stage, out_hbm.at[pl.ds(r * TOPK, BR * TOPK)])
        return 0

    lax.fori_loop(0, RPW // BR, batch, 0)


def _place(ranks, ci):
    mesh = plsc.VectorSubcoreMesh(core_axis_name="c", subcore_axis_name="s")
    f = pl.kernel(
        _place_body,
        out_type=jax.ShapeDtypeStruct((ROWS * TOPK,), jnp.int32),
        mesh=mesh,
        scratch_types=[
            pltpu.VMEM((8 * CAP,), jnp.int32),
            pltpu.VMEM((8 * CAP,), jnp.int32),
            pltpu.VMEM((8 * TOPK,), jnp.int32),
            pltpu.SemaphoreType.DMA,
        ],
        compiler_params=pltpu.CompilerParams(needs_layout_passes=False),
    )
    return f(ranks, ci)


def kernel(x, qr, freqs_cis, wq_b, wk, k_norm_w, k_norm_b, weights_proj):
    b, s, _ = x.shape
    qr2 = qr[0]
    rs = jnp.concatenate([freqs_cis, freqs_cis], axis=-1)
    Hb = _bf(jnp.asarray(_hada128()))
    wqbT = _bf(wq_b.T)

    kx = (x @ wk.T).astype(jnp.float32)
    mu = jnp.mean(kx, axis=-1, keepdims=True)
    var = jnp.mean((kx - mu) ** 2, axis=-1, keepdims=True)
    k = (kx - mu) / jnp.sqrt(var + EPS) * k_norm_w + k_norm_b
    k = k * jnp.concatenate([rs, jnp.ones_like(rs)], axis=-1)[None]
    kh = k[0] @ jnp.asarray(_hada128())
    weights = (x @ weights_proj.T)[0] * (N_HEADS ** -0.5)
    khT = _bf(kh.T)

    n_prep = ROWS // PREP_TILE
    qh = pl.pallas_call(
        _qprep_body,
        grid=(n_prep,),
        in_specs=[
            pl.BlockSpec((PREP_TILE, 512), lambda i: (i, 0)),
            pl.BlockSpec((PREP_TILE, ROPE_DIM), lambda i: (i, 0)),
            pl.BlockSpec((512, 2048), lambda i: (0, 0)),
            pl.BlockSpec((HEAD_DIM, HEAD_DIM), lambda i: (0, 0)),
        ],
        out_specs=pl.BlockSpec((PREP_TILE, 2048), lambda i: (i, 0)),
        out_shape=jax.ShapeDtypeStruct((ROWS, 2048), jnp.float32),
    )(qr2, rs, wqbT, Hb)

    qh_hm = _bf(qh.reshape(ROWS, N_HEADS, HEAD_DIM).transpose(1, 0, 2))

    n_sc = ROWS // SCORE_TILE
    index_scores = pl.pallas_call(
        _score_body,
        grid=(n_sc,),
        in_specs=[
            pl.BlockSpec((N_HEADS, SCORE_TILE, HEAD_DIM), lambda i: (0, i, 0)),
            pl.BlockSpec((HEAD_DIM, ROWS), lambda i: (0, 0)),
            pl.BlockSpec((SCORE_TILE, N_HEADS), lambda i: (i, 0)),
        ],
        out_specs=pl.BlockSpec((SCORE_TILE, ROWS), lambda i: (i, 0)),
        out_shape=jax.ShapeDtypeStruct((ROWS, ROWS), jnp.float32),
    )(qh_hm, khT, weights)

    cand_key_f, cand_idx_f = _select_candidates(index_scores.reshape(-1))
    cand_key = cand_key_f.reshape(ROWS, CAP)

    tri = jnp.triu(jnp.ones((CAP, CAP), jnp.int32), k=1)
    ranks = pl.pallas_call(
        _rank_body,
        grid=(ROWS // 8,),
        in_specs=[
            pl.BlockSpec((8, CAP), lambda i: (i, 0)),
            pl.BlockSpec((CAP, CAP), lambda i: (0, 0)),
        ],
        out_specs=pl.BlockSpec((8, CAP), lambda i: (i, 0)),
        out_shape=jax.ShapeDtypeStruct((ROWS, CAP), jnp.int32),
    )(cand_key, tri)

    out = _place(ranks.reshape(-1), cand_idx_f)
    return out.reshape(1, ROWS, TOPK)
